# baseline (speedup 1.0000x reference)
import functools

import jax
import jax.numpy as jnp
from jax import lax
from jax.experimental import pallas as pl
from jax.experimental.pallas import tpu as pltpu
from jax.experimental.pallas import tpu_sc as plsc

N_E = 1024
E_DIM = 256
HW = 1024
B = 8
ROWS = B * HW

NC = 2
NS = 16
NW = NC * NS
RPW = ROWS // NW
RG = 32
NG = RPW // RG


def _vq_body(z_ref, e_ref, zq_ref, idx_ref):
    zb = z_ref[0]
    emb = e_ref[...]
    z_sq = jnp.sum(zb * zb, axis=0, keepdims=True)
    e_sq = jnp.sum(emb * emb, axis=1, keepdims=True)
    mm = jnp.dot(emb, zb, preferred_element_type=jnp.float32)
    scores = (z_sq + e_sq) - 2.0 * mm
    m = jnp.min(scores, axis=0, keepdims=True)
    row_iota = jax.lax.broadcasted_iota(jnp.int32, scores.shape, 0)
    idx = jnp.min(jnp.where(scores == m, row_iota, N_E), axis=0)
    idx_ref[0, 0] = idx
    col_iota = jax.lax.broadcasted_iota(jnp.int32, (HW, N_E), 1)
    onehot = (col_iota == idx[:, None]).astype(jnp.float32)
    zq_ref[0] = jax.lax.dot_general(
        emb, onehot, (((0,), (1,)), ((), ())),
        preferred_element_type=jnp.float32)


@functools.partial(
    pl.kernel,
    out_type=jax.ShapeDtypeStruct((ROWS * N_E,), jnp.float32),
    mesh=plsc.VectorSubcoreMesh(core_axis_name="c", subcore_axis_name="s"),
    scratch_types=[
        pltpu.VMEM((RPW,), jnp.int32),
        pltpu.VMEM((RG * N_E,), jnp.float32),
    ],
    compiler_params=pltpu.CompilerParams(needs_layout_passes=False),
)
def _sc_onehot(idx_hbm, zrow_hbm, enc_hbm, idx_v, buf_v):
    wid = lax.axis_index("s") * NC + lax.axis_index("c")
    base = wid * RPW
    pltpu.sync_copy(idx_hbm.at[pl.ds(base, RPW)], idx_v)
    pltpu.sync_copy(zrow_hbm, buf_v)
    ones = jnp.full((16,), 1.0, jnp.float32)
    zeros = jnp.zeros((16,), jnp.float32)
    lane = lax.iota(jnp.int32, 16)

    def group(g, carry):
        def patch(j, val):
            cols = idx_v[pl.ds(g * RG + j * 16, 16)]
            flat = (lane + j * 16) * N_E + cols
            plsc.store_scatter(buf_v, [flat], val)
            return val

        lax.fori_loop(0, RG // 16, patch, ones)
        pltpu.sync_copy(buf_v, enc_hbm.at[pl.ds((base + g * RG) * N_E, RG * N_E)])
        lax.fori_loop(0, RG // 16, patch, zeros)
        return carry

    lax.fori_loop(0, NG, group, 0)


@jax.jit
def kernel(z, embedding):
    z3 = z.reshape(B, E_DIM, HW)
    zq, idx = pl.pallas_call(
        _vq_body,
        grid=(B,),
        in_specs=[
            pl.BlockSpec((1, E_DIM, HW), lambda b: (b, 0, 0)),
            pl.BlockSpec((N_E, E_DIM), lambda b: (0, 0)),
        ],
        out_specs=[
            pl.BlockSpec((1, E_DIM, HW), lambda b: (b, 0, 0)),
            pl.BlockSpec((1, 1, HW), lambda b: (b, 0, 0)),
        ],
        out_shape=[
            jax.ShapeDtypeStruct((B, E_DIM, HW), jnp.float32),
            jax.ShapeDtypeStruct((B, 1, HW), jnp.int32),
        ],
    )(z3, embedding)
    idx_flat = idx.reshape(ROWS)
    enc = _sc_onehot(idx_flat, jnp.zeros((RG * N_E,), jnp.float32)).reshape(ROWS, N_E)
    z_q = zq.reshape(B, E_DIM, 32, 32)
    return (z_q, (enc, idx_flat.reshape(ROWS, 1)))


# speedup vs baseline: 1.3419x; 1.3419x over previous
import functools

import jax
import jax.numpy as jnp
from jax import lax
from jax.experimental import pallas as pl
from jax.experimental.pallas import tpu as pltpu
from jax.experimental.pallas import tpu_sc as plsc

N_E = 1024
E_DIM = 256
HW = 1024
B = 8
ROWS = B * HW

NC = 2
NS = 16
NW = NC * NS
RPW = ROWS // NW
RG = 32
NG = RPW // RG


def _vq_body(z_ref, e_ref, es_ref, zq_ref, idx_ref):
    zb = z_ref[0]
    emb = e_ref[...]
    e_sq = es_ref[...]
    zzT = jnp.transpose(zb * zb)
    z_sq = jnp.transpose(jnp.sum(zzT, axis=1)[:, None])
    mm = jnp.dot(emb, zb, preferred_element_type=jnp.float32)
    scores = (z_sq + e_sq) - 2.0 * mm
    m = jnp.min(scores, axis=0, keepdims=True)
    row_iota = jax.lax.broadcasted_iota(jnp.int32, scores.shape, 0)
    idx = jnp.min(jnp.where(scores == m, row_iota, N_E), axis=0)
    idx_ref[0, 0] = idx
    col_iota = jax.lax.broadcasted_iota(jnp.int32, (HW, N_E), 1)
    onehot = (col_iota == idx[:, None]).astype(jnp.float32)
    zq_ref[0] = jax.lax.dot_general(
        emb, onehot, (((0,), (1,)), ((), ())),
        preferred_element_type=jnp.float32)


@functools.partial(
    pl.kernel,
    out_type=jax.ShapeDtypeStruct((ROWS, N_E), jnp.float32),
    mesh=plsc.VectorSubcoreMesh(core_axis_name="c", subcore_axis_name="s"),
    scratch_types=[
        pltpu.VMEM((RPW,), jnp.int32),
        pltpu.VMEM((RG, N_E), jnp.float32),
    ],
    compiler_params=pltpu.CompilerParams(needs_layout_passes=False),
)
def _sc_onehot(idx_hbm, zrow_hbm, enc_hbm, idx_v, buf_v):
    wid = lax.axis_index("s") * NC + lax.axis_index("c")
    base = wid * RPW
    pltpu.sync_copy(idx_hbm.at[pl.ds(base, RPW)], idx_v)
    pltpu.sync_copy(zrow_hbm, buf_v)
    ones = jnp.full((16,), 1.0, jnp.float32)
    zeros = jnp.zeros((16,), jnp.float32)
    lane = lax.iota(jnp.int32, 16)

    def group(g, carry):
        def patch(j, val):
            rows = lane + j * 16
            cols = idx_v[pl.ds(g * RG + j * 16, 16)]
            plsc.store_scatter(buf_v, [rows, cols], val)
            return val

        lax.fori_loop(0, RG // 16, patch, ones)
        pltpu.sync_copy(buf_v, enc_hbm.at[pl.ds(base + g * RG, RG)])
        lax.fori_loop(0, RG // 16, patch, zeros)
        return carry

    lax.fori_loop(0, NG, group, 0)


@jax.jit
def kernel(z, embedding):
    z3 = z.reshape(B, E_DIM, HW)
    e_sq = jnp.sum(embedding ** 2, axis=1)[:, None]
    zq, idx = pl.pallas_call(
        _vq_body,
        grid=(B,),
        in_specs=[
            pl.BlockSpec((1, E_DIM, HW), lambda b: (b, 0, 0)),
            pl.BlockSpec((N_E, E_DIM), lambda b: (0, 0)),
            pl.BlockSpec((N_E, 1), lambda b: (0, 0)),
        ],
        out_specs=[
            pl.BlockSpec((1, E_DIM, HW), lambda b: (b, 0, 0)),
            pl.BlockSpec((1, 1, HW), lambda b: (b, 0, 0)),
        ],
        out_shape=[
            jax.ShapeDtypeStruct((B, E_DIM, HW), jnp.float32),
            jax.ShapeDtypeStruct((B, 1, HW), jnp.int32),
        ],
    )(z3, embedding, e_sq)
    idx_flat = idx.reshape(ROWS)
    enc = _sc_onehot(idx_flat, jnp.zeros((RG, N_E), jnp.float32))
    z_q = zq.reshape(B, E_DIM, 32, 32)
    return (z_q, (enc, idx_flat.reshape(ROWS, 1)))


# speedup vs baseline: 2.1237x; 1.5826x over previous
import functools

import jax
import jax.numpy as jnp
from jax import lax
from jax.experimental import pallas as pl
from jax.experimental.pallas import tpu as pltpu
from jax.experimental.pallas import tpu_sc as plsc

N_E = 1024
E_DIM = 256
HW = 1024
B = 8
ROWS = B * HW

NC = 2
NS = 16
NW = NC * NS
RPW = ROWS // NW
RG = 32
NG = RPW // RG


def _vq_body(z_ref, e_ref, es_ref, zq_ref, enc_ref, idx_ref):
    zb = z_ref[0]
    emb = e_ref[...]
    e_sq = es_ref[...]
    zzT = jnp.transpose(zb * zb)
    z_sq = jnp.transpose(jnp.sum(zzT, axis=1)[:, None])
    mm = jnp.dot(emb, zb, preferred_element_type=jnp.float32)
    scores = (z_sq + e_sq) - 2.0 * mm
    m = jnp.min(scores, axis=0, keepdims=True)
    row_iota = jax.lax.broadcasted_iota(jnp.int32, scores.shape, 0)
    idx = jnp.min(jnp.where(scores == m, row_iota, N_E), axis=0)
    idx_ref[0, 0] = idx
    col_iota = jax.lax.broadcasted_iota(jnp.int32, (HW, N_E), 1)
    onehot = (col_iota == idx[:, None]).astype(jnp.float32)
    enc_ref[...] = onehot
    zq_ref[0] = jax.lax.dot_general(
        emb, onehot, (((0,), (1,)), ((), ())),
        preferred_element_type=jnp.float32)


@functools.partial(
    pl.kernel,
    out_type=jax.ShapeDtypeStruct((ROWS, N_E), jnp.float32),
    mesh=plsc.VectorSubcoreMesh(core_axis_name="c", subcore_axis_name="s"),
    scratch_types=[
        pltpu.VMEM((RPW,), jnp.int32),
        pltpu.VMEM((RG, N_E), jnp.float32),
    ],
    compiler_params=pltpu.CompilerParams(needs_layout_passes=False),
)
def _sc_onehot(idx_hbm, zrow_hbm, enc_hbm, idx_v, buf_v):
    wid = lax.axis_index("s") * NC + lax.axis_index("c")
    base = wid * RPW
    pltpu.sync_copy(idx_hbm.at[pl.ds(base, RPW)], idx_v)
    pltpu.sync_copy(zrow_hbm, buf_v)
    ones = jnp.full((16,), 1.0, jnp.float32)
    zeros = jnp.zeros((16,), jnp.float32)
    lane = lax.iota(jnp.int32, 16)

    def group(g, carry):
        def patch(j, val):
            rows = lane + j * 16
            cols = idx_v[pl.ds(g * RG + j * 16, 16)]
            plsc.store_scatter(buf_v, [rows, cols], val)
            return val

        lax.fori_loop(0, RG // 16, patch, ones)
        pltpu.sync_copy(buf_v, enc_hbm.at[pl.ds(base + g * RG, RG)])
        lax.fori_loop(0, RG // 16, patch, zeros)
        return carry

    lax.fori_loop(0, NG, group, 0)


@jax.jit
def kernel(z, embedding):
    z3 = z.reshape(B, E_DIM, HW)
    e_sq = jnp.sum(embedding ** 2, axis=1)[:, None]
    zq, enc, idx = pl.pallas_call(
        _vq_body,
        grid=(B,),
        in_specs=[
            pl.BlockSpec((1, E_DIM, HW), lambda b: (b, 0, 0)),
            pl.BlockSpec((N_E, E_DIM), lambda b: (0, 0)),
            pl.BlockSpec((N_E, 1), lambda b: (0, 0)),
        ],
        out_specs=[
            pl.BlockSpec((1, E_DIM, HW), lambda b: (b, 0, 0)),
            pl.BlockSpec((HW, N_E), lambda b: (b, 0)),
            pl.BlockSpec((1, 1, HW), lambda b: (b, 0, 0)),
        ],
        out_shape=[
            jax.ShapeDtypeStruct((B, E_DIM, HW), jnp.float32),
            jax.ShapeDtypeStruct((B * HW, N_E), jnp.float32),
            jax.ShapeDtypeStruct((B, 1, HW), jnp.int32),
        ],
    )(z3, embedding, e_sq)
    idx_flat = idx.reshape(ROWS)
    z_q = zq.reshape(B, E_DIM, 32, 32)
    return (z_q, (enc, idx_flat.reshape(ROWS, 1)))


# speedup vs baseline: 2.1855x; 1.0291x over previous
import functools

import jax
import jax.numpy as jnp
from jax import lax
from jax.experimental import pallas as pl
from jax.experimental.pallas import tpu as pltpu
from jax.experimental.pallas import tpu_sc as plsc

N_E = 1024
E_DIM = 256
HW = 1024
B = 8
ROWS = B * HW
BPG = 2
NBG = B // BPG

NC = 2
NS = 16
NW = NC * NS
RPW = ROWS // NW
RG = 32
NG = RPW // RG


def _vq_body(z_ref, e_ref, es_ref, zq_ref, enc_ref, idx_ref):
    emb = e_ref[...]
    e_sq = es_ref[...]
    for b2 in range(BPG):
        zb = z_ref[b2]
        zzT = jnp.transpose(zb * zb)
        z_sq = jnp.transpose(jnp.sum(zzT, axis=1)[:, None])
        mm = jnp.dot(emb, zb, preferred_element_type=jnp.float32)
        scores = (z_sq + e_sq) - 2.0 * mm
        m = jnp.min(scores, axis=0, keepdims=True)
        row_iota = jax.lax.broadcasted_iota(jnp.int32, scores.shape, 0)
        idx = jnp.min(jnp.where(scores == m, row_iota, N_E), axis=0)
        idx_ref[b2, 0] = idx
        col_iota = jax.lax.broadcasted_iota(jnp.int32, (HW, N_E), 1)
        onehot = (col_iota == idx[:, None]).astype(jnp.float32)
        enc_ref[pl.ds(b2 * HW, HW), :] = onehot
        zq_ref[b2] = jax.lax.dot_general(
            emb, onehot, (((0,), (1,)), ((), ())),
            preferred_element_type=jnp.float32)


@functools.partial(
    pl.kernel,
    out_type=jax.ShapeDtypeStruct((ROWS, N_E), jnp.float32),
    mesh=plsc.VectorSubcoreMesh(core_axis_name="c", subcore_axis_name="s"),
    scratch_types=[
        pltpu.VMEM((RPW,), jnp.int32),
        pltpu.VMEM((RG, N_E), jnp.float32),
    ],
    compiler_params=pltpu.CompilerParams(needs_layout_passes=False),
)
def _sc_onehot(idx_hbm, zrow_hbm, enc_hbm, idx_v, buf_v):
    wid = lax.axis_index("s") * NC + lax.axis_index("c")
    base = wid * RPW
    pltpu.sync_copy(idx_hbm.at[pl.ds(base, RPW)], idx_v)
    pltpu.sync_copy(zrow_hbm, buf_v)
    ones = jnp.full((16,), 1.0, jnp.float32)
    zeros = jnp.zeros((16,), jnp.float32)
    lane = lax.iota(jnp.int32, 16)

    def group(g, carry):
        def patch(j, val):
            rows = lane + j * 16
            cols = idx_v[pl.ds(g * RG + j * 16, 16)]
            plsc.store_scatter(buf_v, [rows, cols], val)
            return val

        lax.fori_loop(0, RG // 16, patch, ones)
        pltpu.sync_copy(buf_v, enc_hbm.at[pl.ds(base + g * RG, RG)])
        lax.fori_loop(0, RG // 16, patch, zeros)
        return carry

    lax.fori_loop(0, NG, group, 0)


@jax.jit
def kernel(z, embedding):
    z3 = z.reshape(B, E_DIM, HW)
    e_sq = jnp.sum(embedding ** 2, axis=1)[:, None]
    zq, enc, idx = pl.pallas_call(
        _vq_body,
        grid=(NBG,),
        in_specs=[
            pl.BlockSpec((BPG, E_DIM, HW), lambda b: (b, 0, 0)),
            pl.BlockSpec((N_E, E_DIM), lambda b: (0, 0)),
            pl.BlockSpec((N_E, 1), lambda b: (0, 0)),
        ],
        out_specs=[
            pl.BlockSpec((BPG, E_DIM, HW), lambda b: (b, 0, 0)),
            pl.BlockSpec((BPG * HW, N_E), lambda b: (b, 0)),
            pl.BlockSpec((BPG, 1, HW), lambda b: (b, 0, 0)),
        ],
        out_shape=[
            jax.ShapeDtypeStruct((B, E_DIM, HW), jnp.float32),
            jax.ShapeDtypeStruct((B * HW, N_E), jnp.float32),
            jax.ShapeDtypeStruct((B, 1, HW), jnp.int32),
        ],
    )(z3, embedding, e_sq)
    idx_flat = idx.reshape(ROWS)
    z_q = zq.reshape(B, E_DIM, 32, 32)
    return (z_q, (enc, idx_flat.reshape(ROWS, 1)))
